# Initial kernel scaffold; baseline (speedup 1.0000x reference)
import jax
import jax.numpy as jnp
from jax.experimental import pallas as pl


def kernel(x, edge_index, batch, W1, b1, W2, b2, gamma, beta, Wd, bd):
    raise NotImplementedError("write your pallas kernel here")


# speedup vs baseline: 7.8724x; 7.8724x over previous
import functools
import jax
import jax.numpy as jnp
from jax import lax
from jax.experimental import pallas as pl
from jax.experimental.pallas import tpu as pltpu
from jax.experimental.pallas import tpu_sc as plsc

N = 10000
D = 256
G = 64
OUT = 256
BLK = 1000
NBLK = N // BLK

E = 160000
HD = D // 2
NS = 16
CH = 125
EPT = E // NS
NCH = EPT // CH
NPH = 2
CPP = NCH // NPH
NPAD = 10240
RPT = NPAD // NS
ZR = 80


def _sc_agg_body(hp_hbm, srcr_hbm, dstr_hbm, out_hbm,
                 idx_s, idx_d, gbuf, acc, sems):
    c = lax.axis_index("c")
    s = lax.axis_index("s")

    def _zrow(i, _):
        for j in range(HD // 16):
            gbuf[0, i, pl.ds(j * 16, 16)] = jnp.zeros((16,), jnp.float32)
        return 0
    lax.fori_loop(0, ZR, _zrow, 0)
    zsrc = gbuf.at[0].at[pl.ds(0, ZR)]
    for k in range(RPT // ZR):
        pltpu.sync_copy(zsrc, acc.at[pl.ds(s * RPT + k * ZR, ZR)])

    plsc.subcore_barrier()

    for ph in range(NPH):
        base = s * NCH + ph * CPP
        pltpu.sync_copy(srcr_hbm.at[pl.ds(base, CPP)], idx_s)
        pltpu.sync_copy(dstr_hbm.at[pl.ds(base, CPP)], idx_d)

        pltpu.async_copy(hp_hbm.at[c].at[idx_s.at[0]], gbuf.at[0],
                         sems.at[0])

        def _step(k, _):
            p = lax.rem(k, 2)
            pn = lax.rem(k + 1, 2)

            @pl.when(k + 1 < CPP)
            def _():
                pltpu.async_copy(hp_hbm.at[c].at[idx_s.at[k + 1]],
                                 gbuf.at[pn], sems.at[pn])

            pltpu.make_async_copy(hp_hbm.at[c].at[idx_s.at[k]],
                                  gbuf.at[p], sems.at[p]).wait()
            pltpu.sync_copy(gbuf.at[p], acc.at[idx_d.at[k]], add=True)
            return 0

        lax.fori_loop(0, CPP, _step, 0)

    plsc.subcore_barrier()
    pltpu.sync_copy(acc.at[pl.ds(s * RPT, RPT)],
                    out_hbm.at[c].at[pl.ds(s * RPT, RPT)])


@functools.partial(
    pl.kernel,
    mesh=plsc.VectorSubcoreMesh(core_axis_name="c", subcore_axis_name="s"),
    out_type=jax.ShapeDtypeStruct((2, NPAD, HD), jnp.float32),
    scratch_types=[
        pltpu.VMEM((CPP, CH), jnp.int32),
        pltpu.VMEM((CPP, CH), jnp.int32),
        pltpu.VMEM((2, CH, HD), jnp.float32),
        pltpu.VMEM_SHARED((NPAD, HD), jnp.float32),
        pltpu.SemaphoreType.DMA((2,)),
    ],
)
def _sc_aggregate(hp_hbm, srcr_hbm, dstr_hbm, out_hbm,
                  idx_s, idx_d, gbuf, acc, sems):
    _sc_agg_body(hp_hbm, srcr_hbm, dstr_hbm, out_hbm,
                 idx_s, idx_d, gbuf, acc, sems)


def _mlp_stats_body(aggp_ref, hp_ref, w1_ref, b1_ref, w2_ref, b2_ref,
                    r_ref, sums_ref):
    i = pl.program_id(0)
    m = jnp.concatenate(
        [aggp_ref[0] + hp_ref[0], aggp_ref[1] + hp_ref[1]], axis=-1)
    z = jnp.maximum(
        jnp.dot(m, w1_ref[...], preferred_element_type=jnp.float32)
        + b1_ref[...], 0.0)
    z = jnp.dot(z, w2_ref[...], preferred_element_type=jnp.float32) \
        + b2_ref[...]
    r = jnp.maximum(z, 0.0)
    r_ref[...] = r

    @pl.when(i == 0)
    def _():
        sums_ref[...] = jnp.zeros_like(sums_ref)

    sums_ref[...] += jnp.concatenate(
        [jnp.sum(r, axis=0)[None, :], jnp.sum(r * r, axis=0)[None, :]],
        axis=0)


def _mlp_stats(aggp, hp, w1, b1, w2, b2):
    return pl.pallas_call(
        _mlp_stats_body,
        grid=(NBLK,),
        in_specs=[
            pl.BlockSpec((2, BLK, HD), lambda i: (0, i, 0)),
            pl.BlockSpec((2, BLK, HD), lambda i: (0, i, 0)),
            pl.BlockSpec((D, D), lambda i: (0, 0)),
            pl.BlockSpec((1, D), lambda i: (0, 0)),
            pl.BlockSpec((D, D), lambda i: (0, 0)),
            pl.BlockSpec((1, D), lambda i: (0, 0)),
        ],
        out_specs=[
            pl.BlockSpec((BLK, D), lambda i: (i, 0)),
            pl.BlockSpec((2, D), lambda i: (0, 0)),
        ],
        out_shape=[
            jax.ShapeDtypeStruct((N, D), jnp.float32),
            jax.ShapeDtypeStruct((2, D), jnp.float32),
        ],
    )(aggp, hp, w1, b1, w2, b2)


def _mlp_pool_body(aggp_ref, hp_ref, batch_ref, w1_ref, b1_ref,
                   w2_ref, b2_ref, sums_ref, pool_ref):
    i = pl.program_id(0)
    m = jnp.concatenate(
        [aggp_ref[0] + hp_ref[0], aggp_ref[1] + hp_ref[1]], axis=-1)
    z = jnp.maximum(
        jnp.dot(m, w1_ref[...], preferred_element_type=jnp.float32)
        + b1_ref[...], 0.0)
    z = jnp.dot(z, w2_ref[...], preferred_element_type=jnp.float32) \
        + b2_ref[...]
    r = jnp.maximum(z, 0.0)

    oh_t = jnp.equal(
        lax.broadcasted_iota(jnp.int32, (G, BLK), 0),
        batch_ref[0]).astype(jnp.float32)

    @pl.when(i == 0)
    def _():
        sums_ref[...] = jnp.zeros_like(sums_ref)
        pool_ref[...] = jnp.zeros_like(pool_ref)

    sums_ref[...] += jnp.concatenate(
        [jnp.sum(r, axis=0)[None, :], jnp.sum(r * r, axis=0)[None, :]],
        axis=0)
    pool_ref[...] += jnp.dot(oh_t, r, preferred_element_type=jnp.float32)


def _mlp_pool(aggp, hp, batch2d, w1, b1, w2, b2):
    return pl.pallas_call(
        _mlp_pool_body,
        grid=(NBLK,),
        in_specs=[
            pl.BlockSpec((2, BLK, HD), lambda i: (0, i, 0)),
            pl.BlockSpec((2, BLK, HD), lambda i: (0, i, 0)),
            pl.BlockSpec((1, 1, BLK), lambda i: (i, 0, 0)),
            pl.BlockSpec((D, D), lambda i: (0, 0)),
            pl.BlockSpec((1, D), lambda i: (0, 0)),
            pl.BlockSpec((D, D), lambda i: (0, 0)),
            pl.BlockSpec((1, D), lambda i: (0, 0)),
        ],
        out_specs=[
            pl.BlockSpec((2, D), lambda i: (0, 0)),
            pl.BlockSpec((G, D), lambda i: (0, 0)),
        ],
        out_shape=[
            jax.ShapeDtypeStruct((2, D), jnp.float32),
            jax.ShapeDtypeStruct((G, D), jnp.float32),
        ],
    )(aggp, hp, batch2d, w1, b1, w2, b2)


def _norm_pool_body(r_ref, sums_ref, g_ref, be_ref, batch_ref,
                    hp_ref, pool_ref, cnt_ref):
    i = pl.program_id(0)
    mu = sums_ref[0, :] * (1.0 / N)
    var = sums_ref[1, :] * (1.0 / N) - mu * mu
    rstd = lax.rsqrt(var + 1e-5)
    h = g_ref[...] * (r_ref[...] - mu[None, :]) * rstd[None, :] + be_ref[...]
    hp_ref[0] = h[:, :HD]
    hp_ref[1] = h[:, HD:]

    oh_t = jnp.equal(
        lax.broadcasted_iota(jnp.int32, (G, BLK), 0),
        batch_ref[0]).astype(jnp.float32)

    @pl.when(i == 0)
    def _():
        pool_ref[...] = jnp.zeros_like(pool_ref)
        cnt_ref[...] = jnp.zeros_like(cnt_ref)

    pool_ref[...] += jnp.dot(oh_t, h, preferred_element_type=jnp.float32)
    cnt_ref[...] += jnp.dot(oh_t, jnp.ones((BLK, 128), jnp.float32),
                            preferred_element_type=jnp.float32)


def _norm_pool(r, sums, gamma, beta, batch2d):
    return pl.pallas_call(
        _norm_pool_body,
        grid=(NBLK,),
        in_specs=[
            pl.BlockSpec((BLK, D), lambda i: (i, 0)),
            pl.BlockSpec((2, D), lambda i: (0, 0)),
            pl.BlockSpec((1, D), lambda i: (0, 0)),
            pl.BlockSpec((1, D), lambda i: (0, 0)),
            pl.BlockSpec((1, 1, BLK), lambda i: (i, 0, 0)),
        ],
        out_specs=[
            pl.BlockSpec((2, BLK, HD), lambda i: (0, i, 0)),
            pl.BlockSpec((G, D), lambda i: (0, 0)),
            pl.BlockSpec((G, 128), lambda i: (0, 0)),
        ],
        out_shape=[
            jax.ShapeDtypeStruct((2, N, HD), jnp.float32),
            jax.ShapeDtypeStruct((G, D), jnp.float32),
            jax.ShapeDtypeStruct((G, 128), jnp.float32),
        ],
    )(r, sums, gamma, beta, batch2d)


def _final_body(p1_ref, p2_ref, p3_ref, s3_ref, g3_ref, be3_ref,
                cnt_ref, wd_ref, bd_ref, out_ref):
    cc = cnt_ref[...][:, 0:1]
    inv = 1.0 / jnp.maximum(cc, 1.0)
    mu = s3_ref[0, :] * (1.0 / N)
    var = s3_ref[1, :] * (1.0 / N) - mu * mu
    a = g3_ref[0, :] * lax.rsqrt(var + 1e-5)
    b = be3_ref[0, :] - mu * a
    pool3 = (p3_ref[...] * a[None, :] + cc * b[None, :]) * inv
    p = jnp.concatenate(
        [p1_ref[...] * inv, p2_ref[...] * inv, pool3], axis=1)
    out_ref[...] = jnp.dot(p, wd_ref[...],
                           preferred_element_type=jnp.float32) + bd_ref[...]


def _final(p1, p2, p3, s3, g3, be3, cnt, wd, bd):
    return pl.pallas_call(
        _final_body,
        out_shape=jax.ShapeDtypeStruct((G, OUT), jnp.float32),
    )(p1, p2, p3, s3, g3, be3, cnt, wd, bd)


@jax.jit
def kernel(x, edge_index, batch, W1, b1, W2, b2, gamma, beta, Wd, bd):
    srcr = edge_index[0].reshape(E // CH, CH)
    dstr = edge_index[1].reshape(E // CH, CH)
    batch2d = batch.reshape(NBLK, 1, BLK)
    hp = x.reshape(N, 2, HD).transpose(1, 0, 2)

    aggp = _sc_aggregate(hp, srcr, dstr)
    r, sums = _mlp_stats(aggp, hp, W1[0], b1[0][None, :],
                         W2[0], b2[0][None, :])
    hp, P1, cnt = _norm_pool(r, sums, gamma[0][None, :],
                             beta[0][None, :], batch2d)

    aggp = _sc_aggregate(hp, srcr, dstr)
    r, sums = _mlp_stats(aggp, hp, W1[1], b1[1][None, :],
                         W2[1], b2[1][None, :])
    hp, P2, _ = _norm_pool(r, sums, gamma[1][None, :],
                           beta[1][None, :], batch2d)

    aggp = _sc_aggregate(hp, srcr, dstr)
    sums3, P3raw = _mlp_pool(aggp, hp, batch2d, W1[2], b1[2][None, :],
                             W2[2], b2[2][None, :])

    return _final(P1, P2, P3raw, sums3, gamma[2][None, :],
                  beta[2][None, :], cnt, Wd, bd[None, :])


# speedup vs baseline: 8.0750x; 1.0257x over previous
import functools
import jax
import jax.numpy as jnp
from jax import lax
from jax.experimental import pallas as pl
from jax.experimental.pallas import tpu as pltpu
from jax.experimental.pallas import tpu_sc as plsc

N = 10000
D = 256
G = 64
OUT = 256
BLK = 2000
NBLK = N // BLK

E = 160000
HD = D // 2
NS = 16
CH = 125
EPT = E // NS
NCH = EPT // CH
NPH = 2
CPP = NCH // NPH
NPAD = 10240
RPT = NPAD // NS
ZR = 80


def _sc_agg_body(hp_hbm, srcr_hbm, dstr_hbm, out_hbm,
                 idx_s, idx_d, gbuf, acc, sems):
    c = lax.axis_index("c")
    s = lax.axis_index("s")

    def _zrow(i, _):
        for j in range(HD // 16):
            gbuf[0, i, pl.ds(j * 16, 16)] = jnp.zeros((16,), jnp.float32)
        return 0
    lax.fori_loop(0, ZR, _zrow, 0)
    zsrc = gbuf.at[0].at[pl.ds(0, ZR)]
    for k in range(RPT // ZR):
        pltpu.sync_copy(zsrc, acc.at[pl.ds(s * RPT + k * ZR, ZR)])

    plsc.subcore_barrier()

    for ph in range(NPH):
        base = s * NCH + ph * CPP
        pltpu.sync_copy(srcr_hbm.at[pl.ds(base, CPP)], idx_s)
        pltpu.sync_copy(dstr_hbm.at[pl.ds(base, CPP)], idx_d)

        pltpu.async_copy(hp_hbm.at[c].at[idx_s.at[0]], gbuf.at[0],
                         sems.at[0])

        def _step(k, _):
            p = lax.rem(k, 2)
            pn = lax.rem(k + 1, 2)

            @pl.when(k + 1 < CPP)
            def _():
                pltpu.async_copy(hp_hbm.at[c].at[idx_s.at[k + 1]],
                                 gbuf.at[pn], sems.at[pn])

            pltpu.make_async_copy(hp_hbm.at[c].at[idx_s.at[k]],
                                  gbuf.at[p], sems.at[p]).wait()
            pltpu.sync_copy(gbuf.at[p], acc.at[idx_d.at[k]], add=True)
            return 0

        lax.fori_loop(0, CPP, _step, 0)

    plsc.subcore_barrier()
    pltpu.sync_copy(acc.at[pl.ds(s * RPT, RPT)],
                    out_hbm.at[c].at[pl.ds(s * RPT, RPT)])


@functools.partial(
    pl.kernel,
    mesh=plsc.VectorSubcoreMesh(core_axis_name="c", subcore_axis_name="s"),
    out_type=jax.ShapeDtypeStruct((2, NPAD, HD), jnp.float32),
    scratch_types=[
        pltpu.VMEM((CPP, CH), jnp.int32),
        pltpu.VMEM((CPP, CH), jnp.int32),
        pltpu.VMEM((2, CH, HD), jnp.float32),
        pltpu.VMEM_SHARED((NPAD, HD), jnp.float32),
        pltpu.SemaphoreType.DMA((2,)),
    ],
)
def _sc_aggregate(hp_hbm, srcr_hbm, dstr_hbm, out_hbm,
                  idx_s, idx_d, gbuf, acc, sems):
    _sc_agg_body(hp_hbm, srcr_hbm, dstr_hbm, out_hbm,
                 idx_s, idx_d, gbuf, acc, sems)


def _mlp_stats_body(aggp_ref, hp_ref, w1_ref, b1_ref, w2_ref, b2_ref,
                    r_ref, sums_ref):
    i = pl.program_id(0)
    m = jnp.concatenate(
        [aggp_ref[0] + hp_ref[0], aggp_ref[1] + hp_ref[1]], axis=-1)
    z = jnp.maximum(
        jnp.dot(m, w1_ref[...], preferred_element_type=jnp.float32)
        + b1_ref[...], 0.0)
    z = jnp.dot(z, w2_ref[...], preferred_element_type=jnp.float32) \
        + b2_ref[...]
    r = jnp.maximum(z, 0.0)
    r_ref[...] = r

    @pl.when(i == 0)
    def _():
        sums_ref[...] = jnp.zeros_like(sums_ref)

    sums_ref[...] += jnp.concatenate(
        [jnp.sum(r, axis=0)[None, :], jnp.sum(r * r, axis=0)[None, :]],
        axis=0)


def _mlp_stats(aggp, hp, w1, b1, w2, b2):
    return pl.pallas_call(
        _mlp_stats_body,
        grid=(NBLK,),
        in_specs=[
            pl.BlockSpec((2, BLK, HD), lambda i: (0, i, 0)),
            pl.BlockSpec((2, BLK, HD), lambda i: (0, i, 0)),
            pl.BlockSpec((D, D), lambda i: (0, 0)),
            pl.BlockSpec((1, D), lambda i: (0, 0)),
            pl.BlockSpec((D, D), lambda i: (0, 0)),
            pl.BlockSpec((1, D), lambda i: (0, 0)),
        ],
        out_specs=[
            pl.BlockSpec((BLK, D), lambda i: (i, 0)),
            pl.BlockSpec((2, D), lambda i: (0, 0)),
        ],
        out_shape=[
            jax.ShapeDtypeStruct((N, D), jnp.float32),
            jax.ShapeDtypeStruct((2, D), jnp.float32),
        ],
    )(aggp, hp, w1, b1, w2, b2)


def _mlp_pool_body(aggp_ref, hp_ref, batch_ref, w1_ref, b1_ref,
                   w2_ref, b2_ref, sums_ref, pool_ref):
    i = pl.program_id(0)
    m = jnp.concatenate(
        [aggp_ref[0] + hp_ref[0], aggp_ref[1] + hp_ref[1]], axis=-1)
    z = jnp.maximum(
        jnp.dot(m, w1_ref[...], preferred_element_type=jnp.float32)
        + b1_ref[...], 0.0)
    z = jnp.dot(z, w2_ref[...], preferred_element_type=jnp.float32) \
        + b2_ref[...]
    r = jnp.maximum(z, 0.0)

    oh_t = jnp.equal(
        lax.broadcasted_iota(jnp.int32, (G, BLK), 0),
        batch_ref[0]).astype(jnp.float32)

    @pl.when(i == 0)
    def _():
        sums_ref[...] = jnp.zeros_like(sums_ref)
        pool_ref[...] = jnp.zeros_like(pool_ref)

    sums_ref[...] += jnp.concatenate(
        [jnp.sum(r, axis=0)[None, :], jnp.sum(r * r, axis=0)[None, :]],
        axis=0)
    pool_ref[...] += jnp.dot(oh_t, r, preferred_element_type=jnp.float32)


def _mlp_pool(aggp, hp, batch2d, w1, b1, w2, b2):
    return pl.pallas_call(
        _mlp_pool_body,
        grid=(NBLK,),
        in_specs=[
            pl.BlockSpec((2, BLK, HD), lambda i: (0, i, 0)),
            pl.BlockSpec((2, BLK, HD), lambda i: (0, i, 0)),
            pl.BlockSpec((1, 1, BLK), lambda i: (i, 0, 0)),
            pl.BlockSpec((D, D), lambda i: (0, 0)),
            pl.BlockSpec((1, D), lambda i: (0, 0)),
            pl.BlockSpec((D, D), lambda i: (0, 0)),
            pl.BlockSpec((1, D), lambda i: (0, 0)),
        ],
        out_specs=[
            pl.BlockSpec((2, D), lambda i: (0, 0)),
            pl.BlockSpec((G, D), lambda i: (0, 0)),
        ],
        out_shape=[
            jax.ShapeDtypeStruct((2, D), jnp.float32),
            jax.ShapeDtypeStruct((G, D), jnp.float32),
        ],
    )(aggp, hp, batch2d, w1, b1, w2, b2)


def _norm_pool_body(r_ref, sums_ref, g_ref, be_ref, batch_ref,
                    hp_ref, pool_ref, cnt_ref):
    i = pl.program_id(0)
    mu = sums_ref[0, :] * (1.0 / N)
    var = sums_ref[1, :] * (1.0 / N) - mu * mu
    rstd = lax.rsqrt(var + 1e-5)
    h = g_ref[...] * (r_ref[...] - mu[None, :]) * rstd[None, :] + be_ref[...]
    hp_ref[0] = h[:, :HD]
    hp_ref[1] = h[:, HD:]

    oh_t = jnp.equal(
        lax.broadcasted_iota(jnp.int32, (G, BLK), 0),
        batch_ref[0]).astype(jnp.float32)

    @pl.when(i == 0)
    def _():
        pool_ref[...] = jnp.zeros_like(pool_ref)
        cnt_ref[...] = jnp.zeros_like(cnt_ref)

    pool_ref[...] += jnp.dot(oh_t, h, preferred_element_type=jnp.float32)
    cnt_ref[...] += jnp.dot(oh_t, jnp.ones((BLK, 128), jnp.float32),
                            preferred_element_type=jnp.float32)


def _norm_pool(r, sums, gamma, beta, batch2d):
    return pl.pallas_call(
        _norm_pool_body,
        grid=(NBLK,),
        in_specs=[
            pl.BlockSpec((BLK, D), lambda i: (i, 0)),
            pl.BlockSpec((2, D), lambda i: (0, 0)),
            pl.BlockSpec((1, D), lambda i: (0, 0)),
            pl.BlockSpec((1, D), lambda i: (0, 0)),
            pl.BlockSpec((1, 1, BLK), lambda i: (i, 0, 0)),
        ],
        out_specs=[
            pl.BlockSpec((2, BLK, HD), lambda i: (0, i, 0)),
            pl.BlockSpec((G, D), lambda i: (0, 0)),
            pl.BlockSpec((G, 128), lambda i: (0, 0)),
        ],
        out_shape=[
            jax.ShapeDtypeStruct((2, N, HD), jnp.float32),
            jax.ShapeDtypeStruct((G, D), jnp.float32),
            jax.ShapeDtypeStruct((G, 128), jnp.float32),
        ],
    )(r, sums, gamma, beta, batch2d)


def _final_body(p1_ref, p2_ref, p3_ref, s3_ref, g3_ref, be3_ref,
                cnt_ref, wd_ref, bd_ref, out_ref):
    cc = cnt_ref[...][:, 0:1]
    inv = 1.0 / jnp.maximum(cc, 1.0)
    mu = s3_ref[0, :] * (1.0 / N)
    var = s3_ref[1, :] * (1.0 / N) - mu * mu
    a = g3_ref[0, :] * lax.rsqrt(var + 1e-5)
    b = be3_ref[0, :] - mu * a
    pool3 = (p3_ref[...] * a[None, :] + cc * b[None, :]) * inv
    p = jnp.concatenate(
        [p1_ref[...] * inv, p2_ref[...] * inv, pool3], axis=1)
    out_ref[...] = jnp.dot(p, wd_ref[...],
                           preferred_element_type=jnp.float32) + bd_ref[...]


def _final(p1, p2, p3, s3, g3, be3, cnt, wd, bd):
    return pl.pallas_call(
        _final_body,
        out_shape=jax.ShapeDtypeStruct((G, OUT), jnp.float32),
    )(p1, p2, p3, s3, g3, be3, cnt, wd, bd)


@jax.jit
def kernel(x, edge_index, batch, W1, b1, W2, b2, gamma, beta, Wd, bd):
    srcr = edge_index[0].reshape(E // CH, CH)
    dstr = edge_index[1].reshape(E // CH, CH)
    batch2d = batch.reshape(NBLK, 1, BLK)
    hp = x.reshape(N, 2, HD).transpose(1, 0, 2)

    aggp = _sc_aggregate(hp, srcr, dstr)
    r, sums = _mlp_stats(aggp, hp, W1[0], b1[0][None, :],
                         W2[0], b2[0][None, :])
    hp, P1, cnt = _norm_pool(r, sums, gamma[0][None, :],
                             beta[0][None, :], batch2d)

    aggp = _sc_aggregate(hp, srcr, dstr)
    r, sums = _mlp_stats(aggp, hp, W1[1], b1[1][None, :],
                         W2[1], b2[1][None, :])
    hp, P2, _ = _norm_pool(r, sums, gamma[1][None, :],
                           beta[1][None, :], batch2d)

    aggp = _sc_aggregate(hp, srcr, dstr)
    sums3, P3raw = _mlp_pool(aggp, hp, batch2d, W1[2], b1[2][None, :],
                             W2[2], b2[2][None, :])

    return _final(P1, P2, P3raw, sums3, gamma[2][None, :],
                  beta[2][None, :], cnt, Wd, bd[None, :])


# speedup vs baseline: 8.2698x; 1.0241x over previous
import functools
import jax
import jax.numpy as jnp
from jax import lax
from jax.experimental import pallas as pl
from jax.experimental.pallas import tpu as pltpu
from jax.experimental.pallas import tpu_sc as plsc

N = 10000
D = 256
G = 64
OUT = 256
BLK = 5000
NBLK = N // BLK

E = 160000
HD = D // 2
NS = 16
CH = 125
EPT = E // NS
NCH = EPT // CH
NPH = 2
CPP = NCH // NPH
NPAD = 10240
RPT = NPAD // NS
ZR = 80


def _sc_agg_body(hp_hbm, srcr_hbm, dstr_hbm, out_hbm,
                 idx_s, idx_d, gbuf, acc, sems):
    c = lax.axis_index("c")
    s = lax.axis_index("s")

    def _zrow(i, _):
        for j in range(HD // 16):
            gbuf[0, i, pl.ds(j * 16, 16)] = jnp.zeros((16,), jnp.float32)
        return 0
    lax.fori_loop(0, ZR, _zrow, 0)
    zsrc = gbuf.at[0].at[pl.ds(0, ZR)]
    for k in range(RPT // ZR):
        pltpu.sync_copy(zsrc, acc.at[pl.ds(s * RPT + k * ZR, ZR)])

    plsc.subcore_barrier()

    for ph in range(NPH):
        base = s * NCH + ph * CPP
        pltpu.sync_copy(srcr_hbm.at[pl.ds(base, CPP)], idx_s)
        pltpu.sync_copy(dstr_hbm.at[pl.ds(base, CPP)], idx_d)

        pltpu.async_copy(hp_hbm.at[c].at[idx_s.at[0]], gbuf.at[0],
                         sems.at[0])

        def _step(k, _):
            p = lax.rem(k, 2)
            pn = lax.rem(k + 1, 2)

            @pl.when(k + 1 < CPP)
            def _():
                pltpu.async_copy(hp_hbm.at[c].at[idx_s.at[k + 1]],
                                 gbuf.at[pn], sems.at[pn])

            pltpu.make_async_copy(hp_hbm.at[c].at[idx_s.at[k]],
                                  gbuf.at[p], sems.at[p]).wait()
            pltpu.sync_copy(gbuf.at[p], acc.at[idx_d.at[k]], add=True)
            return 0

        lax.fori_loop(0, CPP, _step, 0)

    plsc.subcore_barrier()
    pltpu.sync_copy(acc.at[pl.ds(s * RPT, RPT)],
                    out_hbm.at[c].at[pl.ds(s * RPT, RPT)])


@functools.partial(
    pl.kernel,
    mesh=plsc.VectorSubcoreMesh(core_axis_name="c", subcore_axis_name="s"),
    out_type=jax.ShapeDtypeStruct((2, NPAD, HD), jnp.float32),
    scratch_types=[
        pltpu.VMEM((CPP, CH), jnp.int32),
        pltpu.VMEM((CPP, CH), jnp.int32),
        pltpu.VMEM((2, CH, HD), jnp.float32),
        pltpu.VMEM_SHARED((NPAD, HD), jnp.float32),
        pltpu.SemaphoreType.DMA((2,)),
    ],
)
def _sc_aggregate(hp_hbm, srcr_hbm, dstr_hbm, out_hbm,
                  idx_s, idx_d, gbuf, acc, sems):
    _sc_agg_body(hp_hbm, srcr_hbm, dstr_hbm, out_hbm,
                 idx_s, idx_d, gbuf, acc, sems)


def _mlp_stats_body(aggp_ref, hp_ref, w1_ref, b1_ref, w2_ref, b2_ref,
                    r_ref, sums_ref):
    i = pl.program_id(0)
    m = jnp.concatenate(
        [aggp_ref[0] + hp_ref[0], aggp_ref[1] + hp_ref[1]], axis=-1)
    z = jnp.maximum(
        jnp.dot(m, w1_ref[...], preferred_element_type=jnp.float32)
        + b1_ref[...], 0.0)
    z = jnp.dot(z, w2_ref[...], preferred_element_type=jnp.float32) \
        + b2_ref[...]
    r = jnp.maximum(z, 0.0)
    r_ref[...] = r

    @pl.when(i == 0)
    def _():
        sums_ref[...] = jnp.zeros_like(sums_ref)

    sums_ref[...] += jnp.concatenate(
        [jnp.sum(r, axis=0)[None, :], jnp.sum(r * r, axis=0)[None, :]],
        axis=0)


def _mlp_stats(aggp, hp, w1, b1, w2, b2):
    return pl.pallas_call(
        _mlp_stats_body,
        grid=(NBLK,),
        in_specs=[
            pl.BlockSpec((2, BLK, HD), lambda i: (0, i, 0)),
            pl.BlockSpec((2, BLK, HD), lambda i: (0, i, 0)),
            pl.BlockSpec((D, D), lambda i: (0, 0)),
            pl.BlockSpec((1, D), lambda i: (0, 0)),
            pl.BlockSpec((D, D), lambda i: (0, 0)),
            pl.BlockSpec((1, D), lambda i: (0, 0)),
        ],
        out_specs=[
            pl.BlockSpec((BLK, D), lambda i: (i, 0)),
            pl.BlockSpec((2, D), lambda i: (0, 0)),
        ],
        out_shape=[
            jax.ShapeDtypeStruct((N, D), jnp.float32),
            jax.ShapeDtypeStruct((2, D), jnp.float32),
        ],
    )(aggp, hp, w1, b1, w2, b2)


def _mlp_pool_body(aggp_ref, hp_ref, batch_ref, w1_ref, b1_ref,
                   w2_ref, b2_ref, sums_ref, pool_ref):
    i = pl.program_id(0)
    m = jnp.concatenate(
        [aggp_ref[0] + hp_ref[0], aggp_ref[1] + hp_ref[1]], axis=-1)
    z = jnp.maximum(
        jnp.dot(m, w1_ref[...], preferred_element_type=jnp.float32)
        + b1_ref[...], 0.0)
    z = jnp.dot(z, w2_ref[...], preferred_element_type=jnp.float32) \
        + b2_ref[...]
    r = jnp.maximum(z, 0.0)

    oh_t = jnp.equal(
        lax.broadcasted_iota(jnp.int32, (G, BLK), 0),
        batch_ref[0]).astype(jnp.float32)

    @pl.when(i == 0)
    def _():
        sums_ref[...] = jnp.zeros_like(sums_ref)
        pool_ref[...] = jnp.zeros_like(pool_ref)

    sums_ref[...] += jnp.concatenate(
        [jnp.sum(r, axis=0)[None, :], jnp.sum(r * r, axis=0)[None, :]],
        axis=0)
    pool_ref[...] += jnp.dot(oh_t, r, preferred_element_type=jnp.float32)


def _mlp_pool(aggp, hp, batch2d, w1, b1, w2, b2):
    return pl.pallas_call(
        _mlp_pool_body,
        grid=(NBLK,),
        in_specs=[
            pl.BlockSpec((2, BLK, HD), lambda i: (0, i, 0)),
            pl.BlockSpec((2, BLK, HD), lambda i: (0, i, 0)),
            pl.BlockSpec((1, 1, BLK), lambda i: (i, 0, 0)),
            pl.BlockSpec((D, D), lambda i: (0, 0)),
            pl.BlockSpec((1, D), lambda i: (0, 0)),
            pl.BlockSpec((D, D), lambda i: (0, 0)),
            pl.BlockSpec((1, D), lambda i: (0, 0)),
        ],
        out_specs=[
            pl.BlockSpec((2, D), lambda i: (0, 0)),
            pl.BlockSpec((G, D), lambda i: (0, 0)),
        ],
        out_shape=[
            jax.ShapeDtypeStruct((2, D), jnp.float32),
            jax.ShapeDtypeStruct((G, D), jnp.float32),
        ],
    )(aggp, hp, batch2d, w1, b1, w2, b2)


def _norm_pool_body(r_ref, sums_ref, g_ref, be_ref, batch_ref,
                    hp_ref, pool_ref, cnt_ref):
    i = pl.program_id(0)
    mu = sums_ref[0, :] * (1.0 / N)
    var = sums_ref[1, :] * (1.0 / N) - mu * mu
    rstd = lax.rsqrt(var + 1e-5)
    h = g_ref[...] * (r_ref[...] - mu[None, :]) * rstd[None, :] + be_ref[...]
    hp_ref[0] = h[:, :HD]
    hp_ref[1] = h[:, HD:]

    oh_t = jnp.equal(
        lax.broadcasted_iota(jnp.int32, (G, BLK), 0),
        batch_ref[0]).astype(jnp.float32)

    @pl.when(i == 0)
    def _():
        pool_ref[...] = jnp.zeros_like(pool_ref)
        cnt_ref[...] = jnp.zeros_like(cnt_ref)

    pool_ref[...] += jnp.dot(oh_t, h, preferred_element_type=jnp.float32)
    cnt_ref[...] += jnp.dot(oh_t, jnp.ones((BLK, 128), jnp.float32),
                            preferred_element_type=jnp.float32)


def _norm_pool(r, sums, gamma, beta, batch2d):
    return pl.pallas_call(
        _norm_pool_body,
        grid=(NBLK,),
        in_specs=[
            pl.BlockSpec((BLK, D), lambda i: (i, 0)),
            pl.BlockSpec((2, D), lambda i: (0, 0)),
            pl.BlockSpec((1, D), lambda i: (0, 0)),
            pl.BlockSpec((1, D), lambda i: (0, 0)),
            pl.BlockSpec((1, 1, BLK), lambda i: (i, 0, 0)),
        ],
        out_specs=[
            pl.BlockSpec((2, BLK, HD), lambda i: (0, i, 0)),
            pl.BlockSpec((G, D), lambda i: (0, 0)),
            pl.BlockSpec((G, 128), lambda i: (0, 0)),
        ],
        out_shape=[
            jax.ShapeDtypeStruct((2, N, HD), jnp.float32),
            jax.ShapeDtypeStruct((G, D), jnp.float32),
            jax.ShapeDtypeStruct((G, 128), jnp.float32),
        ],
    )(r, sums, gamma, beta, batch2d)


def _final_body(p1_ref, p2_ref, p3_ref, s3_ref, g3_ref, be3_ref,
                cnt_ref, wd_ref, bd_ref, out_ref):
    cc = cnt_ref[...][:, 0:1]
    inv = 1.0 / jnp.maximum(cc, 1.0)
    mu = s3_ref[0, :] * (1.0 / N)
    var = s3_ref[1, :] * (1.0 / N) - mu * mu
    a = g3_ref[0, :] * lax.rsqrt(var + 1e-5)
    b = be3_ref[0, :] - mu * a
    pool3 = (p3_ref[...] * a[None, :] + cc * b[None, :]) * inv
    p = jnp.concatenate(
        [p1_ref[...] * inv, p2_ref[...] * inv, pool3], axis=1)
    out_ref[...] = jnp.dot(p, wd_ref[...],
                           preferred_element_type=jnp.float32) + bd_ref[...]


def _final(p1, p2, p3, s3, g3, be3, cnt, wd, bd):
    return pl.pallas_call(
        _final_body,
        out_shape=jax.ShapeDtypeStruct((G, OUT), jnp.float32),
    )(p1, p2, p3, s3, g3, be3, cnt, wd, bd)


@jax.jit
def kernel(x, edge_index, batch, W1, b1, W2, b2, gamma, beta, Wd, bd):
    srcr = edge_index[0].reshape(E // CH, CH)
    dstr = edge_index[1].reshape(E // CH, CH)
    batch2d = batch.reshape(NBLK, 1, BLK)
    hp = x.reshape(N, 2, HD).transpose(1, 0, 2)

    aggp = _sc_aggregate(hp, srcr, dstr)
    r, sums = _mlp_stats(aggp, hp, W1[0], b1[0][None, :],
                         W2[0], b2[0][None, :])
    hp, P1, cnt = _norm_pool(r, sums, gamma[0][None, :],
                             beta[0][None, :], batch2d)

    aggp = _sc_aggregate(hp, srcr, dstr)
    r, sums = _mlp_stats(aggp, hp, W1[1], b1[1][None, :],
                         W2[1], b2[1][None, :])
    hp, P2, _ = _norm_pool(r, sums, gamma[1][None, :],
                           beta[1][None, :], batch2d)

    aggp = _sc_aggregate(hp, srcr, dstr)
    sums3, P3raw = _mlp_pool(aggp, hp, batch2d, W1[2], b1[2][None, :],
                             W2[2], b2[2][None, :])

    return _final(P1, P2, P3raw, sums3, gamma[2][None, :],
                  beta[2][None, :], cnt, Wd, bd[None, :])
